# baseline (speedup 1.0000x reference)
import functools

import jax
import jax.numpy as jnp
from jax import lax
from jax.experimental import pallas as pl
from jax.experimental.pallas import tpu as pltpu
from jax.experimental.pallas import tpu_sc as plsc

_NC = 2
_NS = 16
_NW = _NC * _NS
_B = 128
_RBLK = 512
_FRAC0 = 0.49


def _make_sc_segsum(n, d, n_pad, nb0, nb1):
  rows_per_sub = n_pad // _NS
  nbm = max(nb0, nb1)
  mesh = plsc.VectorSubcoreMesh(core_axis_name="c", subcore_axis_name="s")

  @functools.partial(
      pl.kernel,
      out_type=(
          jax.ShapeDtypeStruct((_NC, n_pad, d), jnp.float32),
          jax.ShapeDtypeStruct((_NC, n_pad), jnp.float32),
      ),
      mesh=mesh,
      scratch_types=[
          pltpu.VMEM((nbm, _B), jnp.int32),
          pltpu.VMEM((nbm, _B), jnp.int32),
          pltpu.VMEM((_B, d), jnp.float32),
          pltpu.VMEM((_B,), jnp.float32),
          pltpu.VMEM_SHARED((n_pad, d), jnp.float32),
          pltpu.VMEM_SHARED((n_pad,), jnp.float32),
          pltpu.SemaphoreType.DMA,
      ],
  )
  def sc_segsum(x_h, src_h, dst_h, zr_h, zd_h, part_h, degp_h,
                src_v, dst_v, rows_v, ones_v, acc_sh, deg_sh, sem):
    c = lax.axis_index("c")
    s = lax.axis_index("s")
    w = c * _NS + s
    nb_w = jnp.where(c == 0, nb0, nb1)

    pltpu.sync_copy(zr_h.at[pl.ds(s * rows_per_sub, rows_per_sub)],
                    acc_sh.at[pl.ds(s * rows_per_sub, rows_per_sub)])

    @pl.when(s == 0)
    def _zero_deg():
      pltpu.sync_copy(zd_h, deg_sh)

    for k in range(_B // 16):
      ones_v[pl.ds(16 * k, 16)] = jnp.ones((16,), jnp.float32)

    pltpu.sync_copy(src_h.at[w], src_v)
    pltpu.sync_copy(dst_h.at[w], dst_v)
    plsc.subcore_barrier()

    def body(i, carry):
      pltpu.async_copy(x_h.at[src_v.at[i]], rows_v, sem).wait()
      pltpu.sync_copy(rows_v, acc_sh.at[dst_v.at[i]], add=True)
      pltpu.sync_copy(ones_v, deg_sh.at[dst_v.at[i]], add=True)
      return carry

    lax.fori_loop(0, nb_w, body, 0)
    plsc.subcore_barrier()

    pltpu.sync_copy(acc_sh.at[pl.ds(s * rows_per_sub, rows_per_sub)],
                    part_h.at[c, pl.ds(s * rows_per_sub, rows_per_sub)])

    @pl.when(s == 0)
    def _write_deg():
      pltpu.sync_copy(deg_sh, degp_h.at[c])

  return sc_segsum


def _tc_finish(p0_ref, p1_ref, d0_ref, d1_ref, w_ref, b_ref, o_ref):
  ssum = p0_ref[...] + p1_ref[...]
  deg = d0_ref[...] + d1_ref[...]
  inv = 1.0 / jnp.maximum(deg, 1.0)
  sn = ssum * inv
  h = jnp.dot(sn, w_ref[...], preferred_element_type=jnp.float32)
  h = h + b_ref[...]
  o_ref[...] = jnp.where(h >= 0.0, h, 0.01 * h)


def _split_batches(arr, pad_value, e, nb0, nb1, nbm):
  e0 = _NS * nb0 * _B
  e_pad = e0 + _NS * nb1 * _B
  p = jnp.pad(arr, (0, e_pad - e), constant_values=pad_value)
  c0 = p[:e0].reshape(_NS, nb0, _B)
  c1 = p[e0:].reshape(_NS, nb1, _B)
  c0 = jnp.pad(c0, ((0, 0), (0, nbm - nb0), (0, 0)),
               constant_values=pad_value)
  c1 = jnp.pad(c1, ((0, 0), (0, nbm - nb1), (0, 0)),
               constant_values=pad_value)
  return jnp.concatenate([c0, c1], axis=0)


def kernel(x, edge_index, W, b):
  n, d = x.shape
  e = edge_index.shape[1]
  nb0 = max(1, int(round(e * _FRAC0 / (_NS * _B))))
  nb1 = max(1, -(-(e - _NS * nb0 * _B) // (_NS * _B)))
  nbm = max(nb0, nb1)
  n_pad = (n // _RBLK + 1) * _RBLK

  src = edge_index[0]
  dst = edge_index[1]
  src_r = _split_batches(src, 0, e, nb0, nb1, nbm)
  dst_r = _split_batches(dst, n, e, nb0, nb1, nbm)
  zrows = jnp.zeros((n_pad, d), jnp.float32)
  zdeg = jnp.zeros((n_pad,), jnp.float32)

  part, degp = _make_sc_segsum(n, d, n_pad, nb0, nb1)(
      x, src_r, dst_r, zrows, zdeg)

  grid = n_pad // _RBLK
  d0 = degp[0].reshape(n_pad, 1)
  d1 = degp[1].reshape(n_pad, 1)
  out_pad = pl.pallas_call(
      _tc_finish,
      grid=(grid,),
      in_specs=[
          pl.BlockSpec((_RBLK, d), lambda i: (i, 0)),
          pl.BlockSpec((_RBLK, d), lambda i: (i, 0)),
          pl.BlockSpec((_RBLK, 1), lambda i: (i, 0)),
          pl.BlockSpec((_RBLK, 1), lambda i: (i, 0)),
          pl.BlockSpec((d, d), lambda i: (0, 0)),
          pl.BlockSpec((1, d), lambda i: (0, 0)),
      ],
      out_specs=pl.BlockSpec((_RBLK, d), lambda i: (i, 0)),
      out_shape=jax.ShapeDtypeStruct((n_pad, d), jnp.float32),
  )(part[0], part[1], d0, d1, W, b.reshape(1, d))
  return out_pad[:n]


# speedup vs baseline: 1.0268x; 1.0268x over previous
import functools

import jax
import jax.numpy as jnp
from jax import lax
from jax.experimental import pallas as pl
from jax.experimental.pallas import tpu as pltpu
from jax.experimental.pallas import tpu_sc as plsc

_NC = 2
_NS = 16
_NW = _NC * _NS
_B = 128
_RBLK = 512
_FRAC0 = 0.50


def _make_sc_segsum(n, d, n_pad, nb0, nb1):
  rows_per_sub = n_pad // _NS
  nbm = max(nb0, nb1)
  mesh = plsc.VectorSubcoreMesh(core_axis_name="c", subcore_axis_name="s")

  @functools.partial(
      pl.kernel,
      out_type=(
          jax.ShapeDtypeStruct((_NC, n_pad, d), jnp.float32),
          jax.ShapeDtypeStruct((_NC, n_pad), jnp.float32),
      ),
      mesh=mesh,
      scratch_types=[
          pltpu.VMEM((nbm, _B), jnp.int32),
          pltpu.VMEM((nbm, _B), jnp.int32),
          pltpu.VMEM((_B, d), jnp.float32),
          pltpu.VMEM((_B,), jnp.float32),
          pltpu.VMEM_SHARED((n_pad, d), jnp.float32),
          pltpu.VMEM_SHARED((n_pad,), jnp.float32),
          pltpu.SemaphoreType.DMA,
          pltpu.SemaphoreType.DMA,
      ],
  )
  def sc_segsum(x_h, src_h, dst_h, zr_h, zd_h, part_h, degp_h,
                src_v, dst_v, rows_v, ones_v, acc_sh, deg_sh, sem, sem_deg):
    c = lax.axis_index("c")
    s = lax.axis_index("s")
    w = c * _NS + s
    nb_w = jnp.where(c == 0, nb0, nb1)

    pltpu.sync_copy(zr_h.at[pl.ds(s * rows_per_sub, rows_per_sub)],
                    acc_sh.at[pl.ds(s * rows_per_sub, rows_per_sub)])

    @pl.when(s == 0)
    def _zero_deg():
      pltpu.sync_copy(zd_h, deg_sh)

    for k in range(_B // 16):
      ones_v[pl.ds(16 * k, 16)] = jnp.ones((16,), jnp.float32)

    pltpu.sync_copy(src_h.at[w], src_v)
    pltpu.sync_copy(dst_h.at[w], dst_v)
    plsc.subcore_barrier()

    def body(i, carry):
      pltpu.async_copy(x_h.at[src_v.at[i]], rows_v, sem).wait()
      pltpu.sync_copy(rows_v, acc_sh.at[dst_v.at[i]], add=True)
      pltpu.async_copy(ones_v, deg_sh.at[dst_v.at[i]], sem_deg, add=True)
      return carry

    lax.fori_loop(0, nb_w, body, 0)

    def drain(i, carry):
      pltpu.make_async_copy(ones_v, deg_sh.at[dst_v.at[0]], sem_deg).wait()
      return carry

    lax.fori_loop(0, nb_w, drain, 0)
    plsc.subcore_barrier()

    pltpu.sync_copy(acc_sh.at[pl.ds(s * rows_per_sub, rows_per_sub)],
                    part_h.at[c, pl.ds(s * rows_per_sub, rows_per_sub)])

    @pl.when(s == 0)
    def _write_deg():
      pltpu.sync_copy(deg_sh, degp_h.at[c])

  return sc_segsum


def _tc_finish(p0_ref, p1_ref, d0_ref, d1_ref, w_ref, b_ref, o_ref):
  ssum = p0_ref[...] + p1_ref[...]
  deg = d0_ref[...] + d1_ref[...]
  inv = 1.0 / jnp.maximum(deg, 1.0)
  sn = ssum * inv
  h = jnp.dot(sn, w_ref[...], preferred_element_type=jnp.float32)
  h = h + b_ref[...]
  o_ref[...] = jnp.where(h >= 0.0, h, 0.01 * h)


def _split_batches(arr, pad_value, e, nb0, nb1, nbm):
  e0 = _NS * nb0 * _B
  e_pad = e0 + _NS * nb1 * _B
  p = jnp.pad(arr, (0, e_pad - e), constant_values=pad_value)
  c0 = p[:e0].reshape(_NS, nb0, _B)
  c1 = p[e0:].reshape(_NS, nb1, _B)
  c0 = jnp.pad(c0, ((0, 0), (0, nbm - nb0), (0, 0)),
               constant_values=pad_value)
  c1 = jnp.pad(c1, ((0, 0), (0, nbm - nb1), (0, 0)),
               constant_values=pad_value)
  return jnp.concatenate([c0, c1], axis=0)


def kernel(x, edge_index, W, b):
  n, d = x.shape
  e = edge_index.shape[1]
  nb0 = max(1, int(round(e * _FRAC0 / (_NS * _B))))
  nb1 = max(1, -(-(e - _NS * nb0 * _B) // (_NS * _B)))
  nbm = max(nb0, nb1)
  n_pad = (n // _RBLK + 1) * _RBLK

  src = edge_index[0]
  dst = edge_index[1]
  src_r = _split_batches(src, 0, e, nb0, nb1, nbm)
  dst_r = _split_batches(dst, n, e, nb0, nb1, nbm)
  zrows = jnp.zeros((n_pad, d), jnp.float32)
  zdeg = jnp.zeros((n_pad,), jnp.float32)

  part, degp = _make_sc_segsum(n, d, n_pad, nb0, nb1)(
      x, src_r, dst_r, zrows, zdeg)

  grid = n_pad // _RBLK
  d0 = degp[0].reshape(n_pad, 1)
  d1 = degp[1].reshape(n_pad, 1)
  out_pad = pl.pallas_call(
      _tc_finish,
      grid=(grid,),
      in_specs=[
          pl.BlockSpec((_RBLK, d), lambda i: (i, 0)),
          pl.BlockSpec((_RBLK, d), lambda i: (i, 0)),
          pl.BlockSpec((_RBLK, 1), lambda i: (i, 0)),
          pl.BlockSpec((_RBLK, 1), lambda i: (i, 0)),
          pl.BlockSpec((d, d), lambda i: (0, 0)),
          pl.BlockSpec((1, d), lambda i: (0, 0)),
      ],
      out_specs=pl.BlockSpec((_RBLK, d), lambda i: (i, 0)),
      out_shape=jax.ShapeDtypeStruct((n_pad, d), jnp.float32),
  )(part[0], part[1], d0, d1, W, b.reshape(1, d))
  return out_pad[:n]


# speedup vs baseline: 1.0893x; 1.0609x over previous
import functools

import jax
import jax.numpy as jnp
from jax import lax
from jax.experimental import pallas as pl
from jax.experimental.pallas import tpu as pltpu
from jax.experimental.pallas import tpu_sc as plsc

_NC = 2
_NS = 16
_NW = _NC * _NS
_B = 128
_RBLK = 512
_FRAC0 = 0.58


def _make_sc_segsum(n, d, n_pad, nb0, nb1):
  rows_per_sub = n_pad // _NS
  nbm = max(nb0, nb1)
  mesh = plsc.VectorSubcoreMesh(core_axis_name="c", subcore_axis_name="s")

  @functools.partial(
      pl.kernel,
      out_type=(
          jax.ShapeDtypeStruct((_NC, n_pad, d), jnp.float32),
          jax.ShapeDtypeStruct((_NC, n_pad), jnp.float32),
      ),
      mesh=mesh,
      scratch_types=[
          pltpu.VMEM((nbm, _B), jnp.int32),
          pltpu.VMEM((nbm, _B), jnp.int32),
          pltpu.VMEM((_B, d), jnp.float32),
          pltpu.VMEM((_B,), jnp.float32),
          pltpu.VMEM_SHARED((n_pad, d), jnp.float32),
          pltpu.VMEM_SHARED((n_pad,), jnp.float32),
          pltpu.SemaphoreType.DMA,
          pltpu.SemaphoreType.DMA,
      ],
  )
  def sc_segsum(x_h, src_h, dst_h, zr_h, zd_h, part_h, degp_h,
                src_v, dst_v, rows_v, ones_v, acc_sh, deg_sh, sem, sem_deg):
    c = lax.axis_index("c")
    s = lax.axis_index("s")
    w = c * _NS + s
    nb_w = jnp.where(c == 0, nb0, nb1)

    pltpu.sync_copy(zr_h.at[pl.ds(s * rows_per_sub, rows_per_sub)],
                    acc_sh.at[pl.ds(s * rows_per_sub, rows_per_sub)])

    @pl.when(s == 0)
    def _zero_deg():
      pltpu.sync_copy(zd_h, deg_sh)

    for k in range(_B // 16):
      ones_v[pl.ds(16 * k, 16)] = jnp.ones((16,), jnp.float32)

    pltpu.sync_copy(src_h.at[w], src_v)
    pltpu.sync_copy(dst_h.at[w], dst_v)
    plsc.subcore_barrier()

    def body(i, carry):
      pltpu.async_copy(x_h.at[src_v.at[i]], rows_v, sem).wait()
      pltpu.sync_copy(rows_v, acc_sh.at[dst_v.at[i]], add=True)
      pltpu.async_copy(ones_v, deg_sh.at[dst_v.at[i]], sem_deg, add=True)
      return carry

    lax.fori_loop(0, nb_w, body, 0)

    def drain(i, carry):
      pltpu.make_async_copy(ones_v, deg_sh.at[dst_v.at[0]], sem_deg).wait()
      return carry

    lax.fori_loop(0, nb_w, drain, 0)
    plsc.subcore_barrier()

    pltpu.sync_copy(acc_sh.at[pl.ds(s * rows_per_sub, rows_per_sub)],
                    part_h.at[c, pl.ds(s * rows_per_sub, rows_per_sub)])

    @pl.when(s == 0)
    def _write_deg():
      pltpu.sync_copy(deg_sh, degp_h.at[c])

  return sc_segsum


def _tc_finish(p0_ref, p1_ref, d0_ref, d1_ref, w_ref, b_ref, o_ref):
  ssum = p0_ref[...] + p1_ref[...]
  deg = d0_ref[...] + d1_ref[...]
  inv = 1.0 / jnp.maximum(deg, 1.0)
  sn = ssum * inv
  h = jnp.dot(sn, w_ref[...], preferred_element_type=jnp.float32)
  h = h + b_ref[...]
  o_ref[...] = jnp.where(h >= 0.0, h, 0.01 * h)


def _split_batches(arr, pad_value, e, nb0, nb1, nbm):
  e0 = _NS * nb0 * _B
  e_pad = e0 + _NS * nb1 * _B
  p = jnp.pad(arr, (0, e_pad - e), constant_values=pad_value)
  c0 = p[:e0].reshape(_NS, nb0, _B)
  c1 = p[e0:].reshape(_NS, nb1, _B)
  c0 = jnp.pad(c0, ((0, 0), (0, nbm - nb0), (0, 0)),
               constant_values=pad_value)
  c1 = jnp.pad(c1, ((0, 0), (0, nbm - nb1), (0, 0)),
               constant_values=pad_value)
  return jnp.concatenate([c0, c1], axis=0)


def kernel(x, edge_index, W, b):
  n, d = x.shape
  e = edge_index.shape[1]
  nb0 = max(1, int(round(e * _FRAC0 / (_NS * _B))))
  nb1 = max(1, -(-(e - _NS * nb0 * _B) // (_NS * _B)))
  nbm = max(nb0, nb1)
  n_pad = (n // _RBLK + 1) * _RBLK

  src = edge_index[0]
  dst = edge_index[1]
  src_r = _split_batches(src, 0, e, nb0, nb1, nbm)
  dst_r = _split_batches(dst, n, e, nb0, nb1, nbm)
  zrows = jnp.zeros((n_pad, d), jnp.float32)
  zdeg = jnp.zeros((n_pad,), jnp.float32)

  part, degp = _make_sc_segsum(n, d, n_pad, nb0, nb1)(
      x, src_r, dst_r, zrows, zdeg)

  grid = n_pad // _RBLK
  d0 = degp[0].reshape(n_pad, 1)
  d1 = degp[1].reshape(n_pad, 1)
  out_pad = pl.pallas_call(
      _tc_finish,
      grid=(grid,),
      in_specs=[
          pl.BlockSpec((_RBLK, d), lambda i: (i, 0)),
          pl.BlockSpec((_RBLK, d), lambda i: (i, 0)),
          pl.BlockSpec((_RBLK, 1), lambda i: (i, 0)),
          pl.BlockSpec((_RBLK, 1), lambda i: (i, 0)),
          pl.BlockSpec((d, d), lambda i: (0, 0)),
          pl.BlockSpec((1, d), lambda i: (0, 0)),
      ],
      out_specs=pl.BlockSpec((_RBLK, d), lambda i: (i, 0)),
      out_shape=jax.ShapeDtypeStruct((n_pad, d), jnp.float32),
  )(part[0], part[1], d0, d1, W, b.reshape(1, d))
  return out_pad[:n]


# speedup vs baseline: 1.1044x; 1.0138x over previous
import functools

import jax
import jax.numpy as jnp
from jax import lax
from jax.experimental import pallas as pl
from jax.experimental.pallas import tpu as pltpu
from jax.experimental.pallas import tpu_sc as plsc

_NC = 2
_NS = 16
_NW = _NC * _NS
_B = 128
_RBLK = 512
_FRAC0 = 0.62


def _make_sc_segsum(n, d, n_pad, nb0, nb1):
  rows_per_sub = n_pad // _NS
  nbm = max(nb0, nb1)
  mesh = plsc.VectorSubcoreMesh(core_axis_name="c", subcore_axis_name="s")

  @functools.partial(
      pl.kernel,
      out_type=(
          jax.ShapeDtypeStruct((_NC, n_pad, d), jnp.float32),
          jax.ShapeDtypeStruct((_NC, n_pad), jnp.float32),
      ),
      mesh=mesh,
      scratch_types=[
          pltpu.VMEM((nbm, _B), jnp.int32),
          pltpu.VMEM((nbm, _B), jnp.int32),
          pltpu.VMEM((_B, d), jnp.float32),
          pltpu.VMEM((_B,), jnp.float32),
          pltpu.VMEM_SHARED((n_pad, d), jnp.float32),
          pltpu.VMEM_SHARED((n_pad,), jnp.float32),
          pltpu.SemaphoreType.DMA,
          pltpu.SemaphoreType.DMA,
      ],
  )
  def sc_segsum(x_h, src_h, dst_h, zr_h, zd_h, part_h, degp_h,
                src_v, dst_v, rows_v, ones_v, acc_sh, deg_sh, sem, sem_deg):
    c = lax.axis_index("c")
    s = lax.axis_index("s")
    w = c * _NS + s
    nb_w = jnp.where(c == 0, nb0, nb1)

    pltpu.sync_copy(zr_h.at[pl.ds(s * rows_per_sub, rows_per_sub)],
                    acc_sh.at[pl.ds(s * rows_per_sub, rows_per_sub)])

    @pl.when(s == 0)
    def _zero_deg():
      pltpu.sync_copy(zd_h, deg_sh)

    for k in range(_B // 16):
      ones_v[pl.ds(16 * k, 16)] = jnp.ones((16,), jnp.float32)

    pltpu.sync_copy(src_h.at[w], src_v)
    pltpu.sync_copy(dst_h.at[w], dst_v)
    plsc.subcore_barrier()

    def body(i, carry):
      pltpu.async_copy(x_h.at[src_v.at[i]], rows_v, sem).wait()
      pltpu.sync_copy(rows_v, acc_sh.at[dst_v.at[i]], add=True)
      pltpu.async_copy(ones_v, deg_sh.at[dst_v.at[i]], sem_deg, add=True)
      return carry

    lax.fori_loop(0, nb_w, body, 0)

    def drain(i, carry):
      pltpu.make_async_copy(ones_v, deg_sh.at[dst_v.at[0]], sem_deg).wait()
      return carry

    lax.fori_loop(0, nb_w, drain, 0)
    plsc.subcore_barrier()

    pltpu.sync_copy(acc_sh.at[pl.ds(s * rows_per_sub, rows_per_sub)],
                    part_h.at[c, pl.ds(s * rows_per_sub, rows_per_sub)])

    @pl.when(s == 0)
    def _write_deg():
      pltpu.sync_copy(deg_sh, degp_h.at[c])

  return sc_segsum


def _tc_finish(p0_ref, p1_ref, d0_ref, d1_ref, w_ref, b_ref, o_ref):
  ssum = p0_ref[...] + p1_ref[...]
  deg = d0_ref[...] + d1_ref[...]
  inv = 1.0 / jnp.maximum(deg, 1.0)
  sn = ssum * inv
  h = jnp.dot(sn, w_ref[...], preferred_element_type=jnp.float32)
  h = h + b_ref[...]
  o_ref[...] = jnp.where(h >= 0.0, h, 0.01 * h)


def _split_batches(arr, pad_value, e, nb0, nb1, nbm):
  e0 = _NS * nb0 * _B
  e_pad = e0 + _NS * nb1 * _B
  p = jnp.pad(arr, (0, e_pad - e), constant_values=pad_value)
  c0 = p[:e0].reshape(_NS, nb0, _B)
  c1 = p[e0:].reshape(_NS, nb1, _B)
  c0 = jnp.pad(c0, ((0, 0), (0, nbm - nb0), (0, 0)),
               constant_values=pad_value)
  c1 = jnp.pad(c1, ((0, 0), (0, nbm - nb1), (0, 0)),
               constant_values=pad_value)
  return jnp.concatenate([c0, c1], axis=0)


def kernel(x, edge_index, W, b):
  n, d = x.shape
  e = edge_index.shape[1]
  nb0 = max(1, int(round(e * _FRAC0 / (_NS * _B))))
  nb1 = max(1, -(-(e - _NS * nb0 * _B) // (_NS * _B)))
  nbm = max(nb0, nb1)
  n_pad = (n // _RBLK + 1) * _RBLK

  src = edge_index[0]
  dst = edge_index[1]
  src_r = _split_batches(src, 0, e, nb0, nb1, nbm)
  dst_r = _split_batches(dst, n, e, nb0, nb1, nbm)
  zrows = jnp.zeros((n_pad, d), jnp.float32)
  zdeg = jnp.zeros((n_pad,), jnp.float32)

  part, degp = _make_sc_segsum(n, d, n_pad, nb0, nb1)(
      x, src_r, dst_r, zrows, zdeg)

  grid = n_pad // _RBLK
  d0 = degp[0].reshape(n_pad, 1)
  d1 = degp[1].reshape(n_pad, 1)
  out_pad = pl.pallas_call(
      _tc_finish,
      grid=(grid,),
      in_specs=[
          pl.BlockSpec((_RBLK, d), lambda i: (i, 0)),
          pl.BlockSpec((_RBLK, d), lambda i: (i, 0)),
          pl.BlockSpec((_RBLK, 1), lambda i: (i, 0)),
          pl.BlockSpec((_RBLK, 1), lambda i: (i, 0)),
          pl.BlockSpec((d, d), lambda i: (0, 0)),
          pl.BlockSpec((1, d), lambda i: (0, 0)),
      ],
      out_specs=pl.BlockSpec((_RBLK, d), lambda i: (i, 0)),
      out_shape=jax.ShapeDtypeStruct((n_pad, d), jnp.float32),
  )(part[0], part[1], d0, d1, W, b.reshape(1, d))
  return out_pad[:n]
